# baseline (speedup 1.0000x reference)
import functools

import jax
import jax.numpy as jnp
from jax import lax
from jax.experimental import pallas as pl
from jax.experimental.pallas import tpu as pltpu
from jax.experimental.pallas import tpu_sc as plsc

N_NODES = 10000
D_FEAT = 128
N_CORES = 2
N_SUBCORES = 16
N_TILES = N_CORES * N_SUBCORES
N_PAD = 10240
CHUNK = 128
EDGES_PER_TILE = 10240
N_CHUNKS = EDGES_PER_TILE // CHUNK
E_PAD = N_TILES * EDGES_PER_TILE
ROWS_PER_TILE = N_PAD // N_SUBCORES


def _make_sc_agg(with_deg: bool):
    mesh = plsc.VectorSubcoreMesh(core_axis_name="c", subcore_axis_name="s")
    out_type = [jax.ShapeDtypeStruct((N_CORES, N_PAD, D_FEAT), jnp.float32)]
    scratch = [
        pltpu.VMEM_SHARED((N_PAD, D_FEAT), jnp.float32),
        pltpu.VMEM((N_CHUNKS // 2, CHUNK), jnp.int32),
        pltpu.VMEM((N_CHUNKS // 2, CHUNK), jnp.int32),
        pltpu.VMEM((CHUNK, D_FEAT), jnp.float32),
        pltpu.VMEM((CHUNK, D_FEAT), jnp.float32),
        pltpu.SemaphoreType.DMA,
        pltpu.SemaphoreType.DMA,
        pltpu.SemaphoreType.DMA,
        pltpu.SemaphoreType.DMA,
        pltpu.SemaphoreType.DMA,
    ]
    if with_deg:
        out_type.append(jax.ShapeDtypeStruct((N_CORES, N_PAD), jnp.float32))
        scratch += [
            pltpu.VMEM_SHARED((N_PAD,), jnp.float32),
            pltpu.VMEM((CHUNK,), jnp.float32),
        ]

    def body(x_hbm, srcp, dstp, z2, z1, ones_hbm, *refs):
        if with_deg:
            (agg_out, deg_out, agg_sh, sidx, didx, rows0, rows1, sem0, sem1,
             ssem0, ssem1, dsem, deg_sh, ones_v) = refs
        else:
            (agg_out, agg_sh, sidx, didx, rows0, rows1, sem0, sem1,
             ssem0, ssem1, dsem) = refs
        rows = (rows0, rows1)
        sems = (sem0, sem1)
        ssems = (ssem0, ssem1)
        c = lax.axis_index("c")
        s = lax.axis_index("s")
        wid = c * N_SUBCORES + s
        row0 = s * ROWS_PER_TILE

        pltpu.sync_copy(z2, agg_sh.at[pl.ds(row0, ROWS_PER_TILE)])
        if with_deg:
            pltpu.sync_copy(z1, deg_sh.at[pl.ds(row0, ROWS_PER_TILE)])
            pltpu.sync_copy(ones_hbm, ones_v)
        plsc.subcore_barrier()

        hc = N_CHUNKS // 2
        for half in range(2):
            pltpu.sync_copy(srcp.at[wid].at[pl.ds(half * hc, hc)], sidx)
            pltpu.sync_copy(dstp.at[wid].at[pl.ds(half * hc, hc)], didx)
            pltpu.async_copy(x_hbm.at[sidx.at[0]], rows[0], sems[0])
            pltpu.async_copy(x_hbm.at[sidx.at[1]], rows[1], sems[1])

            def chunk_step(i, carry):
                for b in range(2):
                    idx = i * 2 + b
                    pltpu.make_async_copy(x_hbm.at[sidx.at[idx]], rows[b],
                                          sems[b]).wait()
                    pltpu.async_copy(rows[b], agg_sh.at[didx.at[idx]],
                                     ssems[b], add=True)
                    if with_deg:
                        pltpu.async_copy(ones_v, deg_sh.at[didx.at[idx]],
                                         dsem, add=True)

                    @pl.when(idx + 2 < hc)
                    def _():
                        pltpu.make_async_copy(rows[b],
                                              agg_sh.at[didx.at[idx]],
                                              ssems[b]).wait()
                        pltpu.async_copy(x_hbm.at[sidx.at[idx + 2]], rows[b],
                                         sems[b])
                return carry

            lax.fori_loop(0, hc // 2, chunk_step, 0)
            for b in range(2):
                pltpu.make_async_copy(rows[b], agg_sh.at[didx.at[hc - 2 + b]],
                                      ssems[b]).wait()
            if with_deg:
                def deg_drain(i, carry):
                    pltpu.make_async_copy(ones_v, deg_sh.at[didx.at[i]],
                                          dsem).wait()
                    return carry
                lax.fori_loop(0, hc, deg_drain, 0)
        plsc.subcore_barrier()

        pltpu.sync_copy(agg_sh.at[pl.ds(row0, ROWS_PER_TILE)],
                        agg_out.at[c].at[pl.ds(row0, ROWS_PER_TILE)])
        if with_deg:
            pltpu.sync_copy(deg_sh.at[pl.ds(row0, ROWS_PER_TILE)],
                            deg_out.at[c].at[pl.ds(row0, ROWS_PER_TILE)])

    return pl.kernel(body, out_type=out_type, mesh=mesh,
                     scratch_types=scratch)


_sc_agg_deg = _make_sc_agg(True)
_sc_agg = _make_sc_agg(False)

BM = 2000


def _tc_layer1(x, aggp, degp3, w_self, w_neigh, b):
    def body(x_r, a_r, d_r, ws_r, wn_r, b_r, o_r):
        deg = d_r[0] + d_r[1]
        dinv = 1.0 / jnp.maximum(deg, 1.0)
        agg = (a_r[0] + a_r[1]) * dinv
        h = (jnp.dot(x_r[...], ws_r[...], preferred_element_type=jnp.float32)
             + jnp.dot(agg, wn_r[...], preferred_element_type=jnp.float32)
             + b_r[...])
        o_r[...] = jnp.maximum(h, 0.0)

    grid = (N_NODES // BM,)
    return pl.pallas_call(
        body,
        grid=grid,
        in_specs=[
            pl.BlockSpec((BM, D_FEAT), lambda i: (i, 0)),
            pl.BlockSpec((N_CORES, BM, D_FEAT), lambda i: (0, i, 0)),
            pl.BlockSpec((N_CORES, BM, 1), lambda i: (0, i, 0)),
            pl.BlockSpec((D_FEAT, D_FEAT), lambda i: (0, 0)),
            pl.BlockSpec((D_FEAT, D_FEAT), lambda i: (0, 0)),
            pl.BlockSpec((1, D_FEAT), lambda i: (0, 0)),
        ],
        out_specs=pl.BlockSpec((BM, D_FEAT), lambda i: (i, 0)),
        out_shape=jax.ShapeDtypeStruct((N_NODES, D_FEAT), jnp.float32),
    )(x, aggp, degp3, w_self, w_neigh, b)


def _tc_layer2_out(h1, aggp, degp3, w_self, w_neigh, b, w_out, b_out):
    def body(x_r, a_r, d_r, ws_r, wn_r, b_r, wo_r, bo_r, o_r):
        deg = d_r[0] + d_r[1]
        dinv = 1.0 / jnp.maximum(deg, 1.0)
        agg = (a_r[0] + a_r[1]) * dinv
        h = (jnp.dot(x_r[...], ws_r[...], preferred_element_type=jnp.float32)
             + jnp.dot(agg, wn_r[...], preferred_element_type=jnp.float32)
             + b_r[...])
        h = jnp.maximum(h, 0.0)
        o_r[...] = (jnp.dot(h, wo_r[...], preferred_element_type=jnp.float32)
                    + bo_r[...])

    grid = (N_NODES // BM,)
    c = w_out.shape[1]
    return pl.pallas_call(
        body,
        grid=grid,
        in_specs=[
            pl.BlockSpec((BM, D_FEAT), lambda i: (i, 0)),
            pl.BlockSpec((N_CORES, BM, D_FEAT), lambda i: (0, i, 0)),
            pl.BlockSpec((N_CORES, BM, 1), lambda i: (0, i, 0)),
            pl.BlockSpec((D_FEAT, D_FEAT), lambda i: (0, 0)),
            pl.BlockSpec((D_FEAT, D_FEAT), lambda i: (0, 0)),
            pl.BlockSpec((1, D_FEAT), lambda i: (0, 0)),
            pl.BlockSpec((D_FEAT, c), lambda i: (0, 0)),
            pl.BlockSpec((1, c), lambda i: (0, 0)),
        ],
        out_specs=pl.BlockSpec((BM, c), lambda i: (i, 0)),
        out_shape=jax.ShapeDtypeStruct((N_NODES, c), jnp.float32),
    )(h1, aggp, degp3, w_self, w_neigh, b, w_out, b_out)


def kernel(features, edge_index, W_self1, W_neigh1, b1,
           W_self2, W_neigh2, b2, W_out, b_out):
    src = edge_index[0]
    dst = edge_index[1]
    e = src.shape[0]
    pad = E_PAD - e
    srcp = jnp.concatenate(
        [src, jnp.zeros((pad,), jnp.int32)]).reshape(
            N_TILES, N_CHUNKS, CHUNK)
    dstp = jnp.concatenate(
        [dst, N_NODES + (jnp.arange(pad, dtype=jnp.int32)
                         % (N_PAD - N_NODES))]).reshape(
            N_TILES, N_CHUNKS, CHUNK)
    z2 = jnp.zeros((ROWS_PER_TILE, D_FEAT), jnp.float32)
    z1 = jnp.zeros((ROWS_PER_TILE,), jnp.float32)
    ones = jnp.ones((CHUNK,), jnp.float32)

    agg1p, degp = _sc_agg_deg(features, srcp, dstp, z2, z1, ones)
    degp3 = degp[:, :N_NODES, None]
    b1r = b1.reshape(1, -1)
    h1 = _tc_layer1(features, agg1p[:, :N_NODES], degp3, W_self1, W_neigh1,
                    b1r)

    (agg2p,) = _sc_agg(h1, srcp, dstp, z2, z1, ones)
    out = _tc_layer2_out(h1, agg2p[:, :N_NODES], degp3, W_self2, W_neigh2,
                         b2.reshape(1, -1), W_out, b_out.reshape(1, -1))
    return out


# speedup vs baseline: 1.0181x; 1.0181x over previous
import functools

import jax
import jax.numpy as jnp
from jax import lax
from jax.experimental import pallas as pl
from jax.experimental.pallas import tpu as pltpu
from jax.experimental.pallas import tpu_sc as plsc

N_NODES = 10000
D_FEAT = 128
N_CORES = 2
N_SUBCORES = 16
N_TILES = N_CORES * N_SUBCORES
N_PAD = 10240
CHUNK = 128
EDGES_PER_TILE = 10240
N_CHUNKS = EDGES_PER_TILE // CHUNK
E_PAD = N_TILES * EDGES_PER_TILE
ROWS_PER_TILE = N_PAD // N_SUBCORES


def _make_sc_agg(with_deg: bool):
    mesh = plsc.VectorSubcoreMesh(core_axis_name="c", subcore_axis_name="s")
    out_type = [jax.ShapeDtypeStruct((N_CORES, N_PAD, D_FEAT), jnp.float32)]
    scratch = [
        pltpu.VMEM_SHARED((N_PAD, D_FEAT), jnp.float32),
        pltpu.VMEM((N_CHUNKS // 2, CHUNK), jnp.int32),
        pltpu.VMEM((N_CHUNKS // 2, CHUNK), jnp.int32),
        pltpu.VMEM((CHUNK, D_FEAT), jnp.float32),
        pltpu.VMEM((CHUNK, D_FEAT), jnp.float32),
        pltpu.SemaphoreType.DMA,
        pltpu.SemaphoreType.DMA,
        pltpu.SemaphoreType.DMA,
        pltpu.SemaphoreType.DMA,
        pltpu.SemaphoreType.DMA,
    ]
    if with_deg:
        out_type.append(jax.ShapeDtypeStruct((N_CORES, N_PAD), jnp.float32))
        scratch += [
            pltpu.VMEM_SHARED((N_PAD,), jnp.float32),
            pltpu.VMEM((CHUNK,), jnp.float32),
        ]

    def body(x_hbm, srcp, dstp, z2, z1, ones_hbm, *refs):
        if with_deg:
            (agg_out, deg_out, agg_sh, sidx, didx, rows0, rows1, sem0, sem1,
             ssem0, ssem1, dsem, deg_sh, ones_v) = refs
        else:
            (agg_out, agg_sh, sidx, didx, rows0, rows1, sem0, sem1,
             ssem0, ssem1, dsem) = refs
        rows = (rows0, rows1)
        sems = (sem0, sem1)
        ssems = (ssem0, ssem1)
        c = lax.axis_index("c")
        s = lax.axis_index("s")
        wid = c * N_SUBCORES + s
        row0 = s * ROWS_PER_TILE

        pltpu.sync_copy(z2.at[pl.ds(row0, ROWS_PER_TILE)],
                        agg_sh.at[pl.ds(row0, ROWS_PER_TILE)])
        if with_deg:
            pltpu.sync_copy(z1.at[pl.ds(row0, ROWS_PER_TILE)],
                            deg_sh.at[pl.ds(row0, ROWS_PER_TILE)])
            pltpu.sync_copy(ones_hbm, ones_v)
        plsc.subcore_barrier()

        hc = N_CHUNKS // 2
        for half in range(2):
            pltpu.sync_copy(srcp.at[wid].at[pl.ds(half * hc, hc)], sidx)
            pltpu.sync_copy(dstp.at[wid].at[pl.ds(half * hc, hc)], didx)
            pltpu.async_copy(x_hbm.at[sidx.at[0]], rows[0], sems[0])
            pltpu.async_copy(x_hbm.at[sidx.at[1]], rows[1], sems[1])

            def chunk_step(i, carry):
                for b in range(2):
                    idx = i * 2 + b
                    pltpu.make_async_copy(x_hbm.at[sidx.at[idx]], rows[b],
                                          sems[b]).wait()
                    pltpu.async_copy(rows[b], agg_sh.at[didx.at[idx]],
                                     ssems[b], add=True)
                    if with_deg:
                        pltpu.async_copy(ones_v, deg_sh.at[didx.at[idx]],
                                         dsem, add=True)

                    @pl.when(idx + 2 < hc)
                    def _():
                        pltpu.make_async_copy(rows[b],
                                              agg_sh.at[didx.at[idx]],
                                              ssems[b]).wait()
                        pltpu.async_copy(x_hbm.at[sidx.at[idx + 2]], rows[b],
                                         sems[b])
                return carry

            lax.fori_loop(0, hc // 2, chunk_step, 0)
            for b in range(2):
                pltpu.make_async_copy(rows[b], agg_sh.at[didx.at[hc - 2 + b]],
                                      ssems[b]).wait()
            if with_deg:
                def deg_drain(i, carry):
                    pltpu.make_async_copy(ones_v, deg_sh.at[didx.at[i]],
                                          dsem).wait()
                    return carry
                lax.fori_loop(0, hc, deg_drain, 0)
        plsc.subcore_barrier()

        pltpu.sync_copy(agg_sh.at[pl.ds(row0, ROWS_PER_TILE)],
                        agg_out.at[c].at[pl.ds(row0, ROWS_PER_TILE)])
        if with_deg:
            pltpu.sync_copy(deg_sh.at[pl.ds(row0, ROWS_PER_TILE)],
                            deg_out.at[c].at[pl.ds(row0, ROWS_PER_TILE)])

    return pl.kernel(body, out_type=out_type, mesh=mesh,
                     scratch_types=scratch)


_sc_agg_deg = _make_sc_agg(True)
_sc_agg = _make_sc_agg(False)

BM = 2000


def _tc_layer1(x, aggp, degp3, w_self, w_neigh, b):
    def body(x_r, a_r, d_r, ws_r, wn_r, b_r, o_r):
        deg = d_r[0] + d_r[1]
        dinv = 1.0 / jnp.maximum(deg, 1.0)
        agg = (a_r[0] + a_r[1]) * dinv
        h = (jnp.dot(x_r[...], ws_r[...], preferred_element_type=jnp.float32)
             + jnp.dot(agg, wn_r[...], preferred_element_type=jnp.float32)
             + b_r[...])
        o_r[...] = jnp.maximum(h, 0.0)

    grid = (N_NODES // BM,)
    return pl.pallas_call(
        body,
        grid=grid,
        in_specs=[
            pl.BlockSpec((BM, D_FEAT), lambda i: (i, 0)),
            pl.BlockSpec((N_CORES, BM, D_FEAT), lambda i: (0, i, 0)),
            pl.BlockSpec((N_CORES, BM, 1), lambda i: (0, i, 0)),
            pl.BlockSpec((D_FEAT, D_FEAT), lambda i: (0, 0)),
            pl.BlockSpec((D_FEAT, D_FEAT), lambda i: (0, 0)),
            pl.BlockSpec((1, D_FEAT), lambda i: (0, 0)),
        ],
        out_specs=pl.BlockSpec((BM, D_FEAT), lambda i: (i, 0)),
        out_shape=jax.ShapeDtypeStruct((N_NODES, D_FEAT), jnp.float32),
    )(x, aggp, degp3, w_self, w_neigh, b)


def _tc_layer2_out(h1, aggp, degp3, w_self, w_neigh, b, w_out, b_out):
    def body(x_r, a_r, d_r, ws_r, wn_r, b_r, wo_r, bo_r, o_r):
        deg = d_r[0] + d_r[1]
        dinv = 1.0 / jnp.maximum(deg, 1.0)
        agg = (a_r[0] + a_r[1]) * dinv
        h = (jnp.dot(x_r[...], ws_r[...], preferred_element_type=jnp.float32)
             + jnp.dot(agg, wn_r[...], preferred_element_type=jnp.float32)
             + b_r[...])
        h = jnp.maximum(h, 0.0)
        o_r[...] = (jnp.dot(h, wo_r[...], preferred_element_type=jnp.float32)
                    + bo_r[...])

    grid = (N_NODES // BM,)
    c = w_out.shape[1]
    return pl.pallas_call(
        body,
        grid=grid,
        in_specs=[
            pl.BlockSpec((BM, D_FEAT), lambda i: (i, 0)),
            pl.BlockSpec((N_CORES, BM, D_FEAT), lambda i: (0, i, 0)),
            pl.BlockSpec((N_CORES, BM, 1), lambda i: (0, i, 0)),
            pl.BlockSpec((D_FEAT, D_FEAT), lambda i: (0, 0)),
            pl.BlockSpec((D_FEAT, D_FEAT), lambda i: (0, 0)),
            pl.BlockSpec((1, D_FEAT), lambda i: (0, 0)),
            pl.BlockSpec((D_FEAT, c), lambda i: (0, 0)),
            pl.BlockSpec((1, c), lambda i: (0, 0)),
        ],
        out_specs=pl.BlockSpec((BM, c), lambda i: (i, 0)),
        out_shape=jax.ShapeDtypeStruct((N_NODES, c), jnp.float32),
    )(h1, aggp, degp3, w_self, w_neigh, b, w_out, b_out)


def kernel(features, edge_index, W_self1, W_neigh1, b1,
           W_self2, W_neigh2, b2, W_out, b_out):
    src = edge_index[0]
    dst = edge_index[1]
    e = src.shape[0]
    pad = E_PAD - e
    srcp = jnp.concatenate(
        [src, jnp.zeros((pad,), jnp.int32)]).reshape(
            N_TILES, N_CHUNKS, CHUNK)
    dstp = jnp.concatenate(
        [dst, N_NODES + (jnp.arange(pad, dtype=jnp.int32)
                         % (N_PAD - N_NODES))]).reshape(
            N_TILES, N_CHUNKS, CHUNK)
    z2 = jnp.zeros((N_PAD, D_FEAT), jnp.float32)
    z1 = jnp.zeros((N_PAD,), jnp.float32)
    ones = jnp.ones((CHUNK,), jnp.float32)

    agg1p, degp = _sc_agg_deg(features, srcp, dstp, z2, z1, ones)
    degp3 = degp[:, :N_NODES, None]
    b1r = b1.reshape(1, -1)
    h1 = _tc_layer1(features, agg1p[:, :N_NODES], degp3, W_self1, W_neigh1,
                    b1r)

    (agg2p,) = _sc_agg(h1, srcp, dstp, z2, z1, ones)
    out = _tc_layer2_out(h1, agg2p[:, :N_NODES], degp3, W_self2, W_neigh2,
                         b2.reshape(1, -1), W_out, b_out.reshape(1, -1))
    return out


# speedup vs baseline: 1.0183x; 1.0002x over previous
import functools

import jax
import jax.numpy as jnp
from jax import lax
from jax.experimental import pallas as pl
from jax.experimental.pallas import tpu as pltpu
from jax.experimental.pallas import tpu_sc as plsc

N_NODES = 10000
D_FEAT = 128
N_CORES = 2
N_SUBCORES = 16
N_TILES = N_CORES * N_SUBCORES
N_PAD = 10240
CHUNK = 128
EDGES_PER_TILE = 10240
N_CHUNKS = EDGES_PER_TILE // CHUNK
E_PAD = N_TILES * EDGES_PER_TILE
ROWS_PER_TILE = N_PAD // N_SUBCORES
SPLIT = 4


def _make_sc_agg(with_deg: bool):
    mesh = plsc.VectorSubcoreMesh(core_axis_name="c", subcore_axis_name="s")
    out_type = [jax.ShapeDtypeStruct((N_CORES, N_PAD, D_FEAT), jnp.float32)]
    scratch = [
        pltpu.VMEM_SHARED((N_PAD, D_FEAT), jnp.float32),
        pltpu.VMEM((N_CHUNKS // 2, CHUNK), jnp.int32),
        pltpu.VMEM((N_CHUNKS // 2, CHUNK), jnp.int32),
        pltpu.VMEM((CHUNK, D_FEAT), jnp.float32),
        pltpu.VMEM((CHUNK, D_FEAT), jnp.float32),
        pltpu.SemaphoreType.DMA,
        pltpu.SemaphoreType.DMA,
        pltpu.SemaphoreType.DMA,
        pltpu.SemaphoreType.DMA,
        pltpu.SemaphoreType.DMA,
    ]
    if with_deg:
        out_type.append(jax.ShapeDtypeStruct((N_CORES, N_PAD), jnp.float32))
        scratch += [
            pltpu.VMEM_SHARED((N_PAD,), jnp.float32),
            pltpu.VMEM((CHUNK,), jnp.float32),
        ]

    def body(x_hbm, srcp, dstp, z2, z1, ones_hbm, *refs):
        if with_deg:
            (agg_out, deg_out, agg_sh, sidx, didx, rows0, rows1, sem0, sem1,
             ssem0, ssem1, dsem, deg_sh, ones_v) = refs
        else:
            (agg_out, agg_sh, sidx, didx, rows0, rows1, sem0, sem1,
             ssem0, ssem1, dsem) = refs
        rows = (rows0, rows1)
        sems = (sem0, sem1)
        ssems = (ssem0, ssem1)
        c = lax.axis_index("c")
        s = lax.axis_index("s")
        wid = c * N_SUBCORES + s
        row0 = s * ROWS_PER_TILE

        pltpu.sync_copy(z2.at[pl.ds(row0, ROWS_PER_TILE)],
                        agg_sh.at[pl.ds(row0, ROWS_PER_TILE)])
        if with_deg:
            pltpu.sync_copy(z1.at[pl.ds(row0, ROWS_PER_TILE)],
                            deg_sh.at[pl.ds(row0, ROWS_PER_TILE)])
            pltpu.sync_copy(ones_hbm, ones_v)
        plsc.subcore_barrier()

        hc = N_CHUNKS // 2
        for half in range(2):
            pltpu.sync_copy(srcp.at[wid].at[pl.ds(half * hc, hc)], sidx)
            pltpu.sync_copy(dstp.at[wid].at[pl.ds(half * hc, hc)], didx)
            def start_gather(idx, b):
                for k in range(SPLIT):
                    sub = pl.ds(k * (CHUNK // SPLIT), CHUNK // SPLIT)
                    pltpu.async_copy(x_hbm.at[sidx.at[idx].at[sub]],
                                     rows[b].at[sub], sems[b])

            def wait_gather(idx, b):
                for k in range(SPLIT):
                    sub = pl.ds(k * (CHUNK // SPLIT), CHUNK // SPLIT)
                    pltpu.make_async_copy(x_hbm.at[sidx.at[idx].at[sub]],
                                          rows[b].at[sub], sems[b]).wait()

            start_gather(0, 0)
            start_gather(1, 1)

            def chunk_step(i, carry):
                for b in range(2):
                    idx = i * 2 + b
                    wait_gather(idx, b)
                    pltpu.async_copy(rows[b], agg_sh.at[didx.at[idx]],
                                     ssems[b], add=True)
                    if with_deg:
                        pltpu.async_copy(ones_v, deg_sh.at[didx.at[idx]],
                                         dsem, add=True)

                    @pl.when(idx + 2 < hc)
                    def _():
                        pltpu.make_async_copy(rows[b],
                                              agg_sh.at[didx.at[idx]],
                                              ssems[b]).wait()
                        start_gather(idx + 2, b)
                return carry

            lax.fori_loop(0, hc // 2, chunk_step, 0)
            for b in range(2):
                pltpu.make_async_copy(rows[b], agg_sh.at[didx.at[hc - 2 + b]],
                                      ssems[b]).wait()
            if with_deg:
                def deg_drain(i, carry):
                    pltpu.make_async_copy(ones_v, deg_sh.at[didx.at[i]],
                                          dsem).wait()
                    return carry
                lax.fori_loop(0, hc, deg_drain, 0)
        plsc.subcore_barrier()

        pltpu.sync_copy(agg_sh.at[pl.ds(row0, ROWS_PER_TILE)],
                        agg_out.at[c].at[pl.ds(row0, ROWS_PER_TILE)])
        if with_deg:
            pltpu.sync_copy(deg_sh.at[pl.ds(row0, ROWS_PER_TILE)],
                            deg_out.at[c].at[pl.ds(row0, ROWS_PER_TILE)])

    return pl.kernel(body, out_type=out_type, mesh=mesh,
                     scratch_types=scratch)


_sc_agg_deg = _make_sc_agg(True)
_sc_agg = _make_sc_agg(False)

BM = 2000


def _tc_layer1(x, aggp, degp3, w_self, w_neigh, b):
    def body(x_r, a_r, d_r, ws_r, wn_r, b_r, o_r):
        deg = d_r[0] + d_r[1]
        dinv = 1.0 / jnp.maximum(deg, 1.0)
        agg = (a_r[0] + a_r[1]) * dinv
        h = (jnp.dot(x_r[...], ws_r[...], preferred_element_type=jnp.float32)
             + jnp.dot(agg, wn_r[...], preferred_element_type=jnp.float32)
             + b_r[...])
        o_r[...] = jnp.maximum(h, 0.0)

    grid = (N_NODES // BM,)
    return pl.pallas_call(
        body,
        grid=grid,
        in_specs=[
            pl.BlockSpec((BM, D_FEAT), lambda i: (i, 0)),
            pl.BlockSpec((N_CORES, BM, D_FEAT), lambda i: (0, i, 0)),
            pl.BlockSpec((N_CORES, BM, 1), lambda i: (0, i, 0)),
            pl.BlockSpec((D_FEAT, D_FEAT), lambda i: (0, 0)),
            pl.BlockSpec((D_FEAT, D_FEAT), lambda i: (0, 0)),
            pl.BlockSpec((1, D_FEAT), lambda i: (0, 0)),
        ],
        out_specs=pl.BlockSpec((BM, D_FEAT), lambda i: (i, 0)),
        out_shape=jax.ShapeDtypeStruct((N_NODES, D_FEAT), jnp.float32),
    )(x, aggp, degp3, w_self, w_neigh, b)


def _tc_layer2_out(h1, aggp, degp3, w_self, w_neigh, b, w_out, b_out):
    def body(x_r, a_r, d_r, ws_r, wn_r, b_r, wo_r, bo_r, o_r):
        deg = d_r[0] + d_r[1]
        dinv = 1.0 / jnp.maximum(deg, 1.0)
        agg = (a_r[0] + a_r[1]) * dinv
        h = (jnp.dot(x_r[...], ws_r[...], preferred_element_type=jnp.float32)
             + jnp.dot(agg, wn_r[...], preferred_element_type=jnp.float32)
             + b_r[...])
        h = jnp.maximum(h, 0.0)
        o_r[...] = (jnp.dot(h, wo_r[...], preferred_element_type=jnp.float32)
                    + bo_r[...])

    grid = (N_NODES // BM,)
    c = w_out.shape[1]
    return pl.pallas_call(
        body,
        grid=grid,
        in_specs=[
            pl.BlockSpec((BM, D_FEAT), lambda i: (i, 0)),
            pl.BlockSpec((N_CORES, BM, D_FEAT), lambda i: (0, i, 0)),
            pl.BlockSpec((N_CORES, BM, 1), lambda i: (0, i, 0)),
            pl.BlockSpec((D_FEAT, D_FEAT), lambda i: (0, 0)),
            pl.BlockSpec((D_FEAT, D_FEAT), lambda i: (0, 0)),
            pl.BlockSpec((1, D_FEAT), lambda i: (0, 0)),
            pl.BlockSpec((D_FEAT, c), lambda i: (0, 0)),
            pl.BlockSpec((1, c), lambda i: (0, 0)),
        ],
        out_specs=pl.BlockSpec((BM, c), lambda i: (i, 0)),
        out_shape=jax.ShapeDtypeStruct((N_NODES, c), jnp.float32),
    )(h1, aggp, degp3, w_self, w_neigh, b, w_out, b_out)


def kernel(features, edge_index, W_self1, W_neigh1, b1,
           W_self2, W_neigh2, b2, W_out, b_out):
    src = edge_index[0]
    dst = edge_index[1]
    e = src.shape[0]
    pad = E_PAD - e
    srcp = jnp.concatenate(
        [src, jnp.zeros((pad,), jnp.int32)]).reshape(
            N_TILES, N_CHUNKS, CHUNK)
    dstp = jnp.concatenate(
        [dst, N_NODES + (jnp.arange(pad, dtype=jnp.int32)
                         % (N_PAD - N_NODES))]).reshape(
            N_TILES, N_CHUNKS, CHUNK)
    z2 = jnp.zeros((N_PAD, D_FEAT), jnp.float32)
    z1 = jnp.zeros((N_PAD,), jnp.float32)
    ones = jnp.ones((CHUNK,), jnp.float32)

    agg1p, degp = _sc_agg_deg(features, srcp, dstp, z2, z1, ones)
    degp3 = degp[:, :N_NODES, None]
    b1r = b1.reshape(1, -1)
    h1 = _tc_layer1(features, agg1p[:, :N_NODES], degp3, W_self1, W_neigh1,
                    b1r)

    (agg2p,) = _sc_agg(h1, srcp, dstp, z2, z1, ones)
    out = _tc_layer2_out(h1, agg2p[:, :N_NODES], degp3, W_self2, W_neigh2,
                         b2.reshape(1, -1), W_out, b_out.reshape(1, -1))
    return out
